# baseline (speedup 1.0000x reference)
import dataclasses
import functools

import jax
import jax.numpy as jnp
from jax import lax
from jax.experimental import pallas as pl
from jax.experimental.pallas import tpu as pltpu
from jax.experimental.pallas import tpu_sc as plsc

B = 16384
N = 100000
D = 256
K = 16
A = 64

NW = 32
CPW = B // NW
CHUNK = 64
NCHUNK = CPW // CHUNK
GB = 8
GROWS = GB * K
NBATCH = CHUNK // GB

_NEG = -1e30


def _tree_sum(vals):
    vals = list(vals)
    while len(vals) > 1:
        nxt = [vals[i] + vals[i + 1] for i in range(0, len(vals) - 1, 2)]
        if len(vals) % 2:
            nxt.append(vals[-1])
        vals = nxt
    return vals[0]


def _pack_rows(x):
    u = jax.lax.bitcast_convert_type(x, jnp.int32)
    t = u + 0x7FFF + ((u >> 16) & 1)
    bf = t >> 16
    lo = bf[:, :128] & 0xFFFF
    hi = bf[:, 128:] << 16
    return hi | lo


def _tc_pack_body(e_ref, t_ref):
    t_ref[...] = _pack_rows(e_ref[...])


def _tc_packpre_body(e_ref, cb_ref, w_ref, wq_ref, wkt_ref,
                     t_ref, p_ref, slog_ref):
    i = pl.program_id(0)

    @pl.when(i < 25)
    def _():
        t_ref[...] = _pack_rows(e_ref[...])

    @pl.when(i >= 25)
    def _():
        q = jnp.dot(cb_ref[...], wq_ref[...],
                    preferred_element_type=jnp.float32)
        p = jnp.dot(q, wkt_ref[...], preferred_element_type=jnp.float32)
        p_ref[...] = _pack_rows(p)
        w = w_ref[...]
        slog_ref[...] = jnp.where(w < 1e-6, _NEG,
                                  jnp.log(jnp.maximum(w, 1e-6)))


def _tc_pre_body(cb_ref, w_ref, wq_ref, wkt_ref, p_ref, slog_ref):
    q = jnp.dot(cb_ref[...], wq_ref[...], preferred_element_type=jnp.float32)
    p = jnp.dot(q, wkt_ref[...], preferred_element_type=jnp.float32)
    p_ref[...] = _pack_rows(p)
    w = w_ref[...]
    slog_ref[...] = jnp.where(w < 1e-6, _NEG, jnp.log(jnp.maximum(w, 1e-6)))


def _tc_post_body(cb_ref, wn_ref, wg1_ref, wg2_ref, bg_ref, o_ref):
    cb = cb_ref[...]
    wn = wn_ref[...]
    z = (jnp.dot(cb, wg1_ref[...], preferred_element_type=jnp.float32)
         + jnp.dot(wn, wg2_ref[...], preferred_element_type=jnp.float32)
         + bg_ref[...])
    g = jax.nn.sigmoid(z)
    o_ref[...] = g * cb + (1.0 - g) * wn


def _sc_attention(table, idx2, p, slog, b_off, b_cnt):
    cpw = b_cnt // NW
    nchunk = cpw // CHUNK
    mesh = plsc.VectorSubcoreMesh(core_axis_name="c", subcore_axis_name="s")
    cp = pltpu.CompilerParams()
    if "needs_layout_passes" in pltpu.CompilerParams.__dataclass_fields__:
        cp = dataclasses.replace(cp, needs_layout_passes=False)

    @functools.partial(
        pl.kernel,
        out_type=jax.ShapeDtypeStruct((b_cnt, D), jnp.float32),
        mesh=mesh,
        compiler_params=cp,
        scratch_types=[
            pltpu.VMEM((2, NBATCH, GROWS), jnp.int32),
            pltpu.VMEM((2, CHUNK, 128), jnp.int32),
            pltpu.VMEM((2, CHUNK, K), jnp.float32),
            pltpu.VMEM((2, CHUNK, D), jnp.float32),
            pltpu.VMEM((GROWS, 128), jnp.int32),
            pltpu.VMEM((GROWS, 128), jnp.int32),
            pltpu.VMEM((GROWS, 16), jnp.float32),
            pltpu.SemaphoreType.DMA,
            pltpu.SemaphoreType.DMA,
            pltpu.SemaphoreType.DMA,
            pltpu.SemaphoreType.DMA,
        ],
    )
    def sc_kernel(table_hbm, idx_hbm, p_hbm, slog_hbm, wn_hbm,
                  idx_v, p_v, slog_v, out_v, bufA, bufB, acc_v,
                  semA, semB, semS, semO):
        cid = lax.axis_index("c")
        sid = lax.axis_index("s")
        wid = sid * 2 + cid
        lane = lax.broadcasted_iota(jnp.int32, (K,), 0)
        cols = [jnp.full((16,), d_, jnp.int32) for d_ in range(16)]

        def compute_batch(g, buf, sel):
            @plsc.parallel_loop(0, GB, unroll=2)
            def _t(t):
                tl = g * GB + t
                r0 = t * K
                a0 = t * K
                pchb = [plsc.bitcast(p_v[sel, tl, pl.ds(gr * 16, 16)],
                                     jnp.bfloat16) for gr in range(8)]
                for j in range(K):
                    pp = [pchb[gr]
                          * plsc.bitcast(buf[r0 + j, pl.ds(gr * 16, 16)],
                                         jnp.bfloat16)
                          for gr in range(8)]
                    prods = []
                    for q2 in range(4):
                        ea, eb = plsc.unpack(
                            pp[2 * q2] + pp[2 * q2 + 1],
                            format=plsc.PackFormat.INTERLEAVED)
                        prods.append(ea)
                        prods.append(eb)
                    acc_v[a0 + j, :] = _tree_sum(prods)
                tot = _tree_sum(
                    [plsc.load_gather(acc_v, [lane + a0, cols[d_]])
                     for d_ in range(16)])
                s = slog_v[sel, tl, :] + tot
                e = jnp.exp(s)
                den = jnp.sum(e)
                den = jnp.where(den > 0.0, den, 1.0)
                attn = e / den
                ajb = []
                for j in range(K):
                    bcast = jnp.broadcast_to(attn[j], (16,))
                    ajb.append(plsc.pack(bcast, bcast,
                                         format=plsc.PackFormat.INTERLEAVED))
                for gr in range(8):
                    fa, fb = None, None
                    for q4 in range(4):
                        pr = None
                        for j in range(4 * q4, 4 * q4 + 4):
                            term = ajb[j] * plsc.bitcast(
                                buf[r0 + j, pl.ds(gr * 16, 16)],
                                jnp.bfloat16)
                            pr = term if pr is None else pr + term
                        ea, eb = plsc.unpack(
                            pr, format=plsc.PackFormat.INTERLEAVED)
                        fa = ea if fa is None else fa + ea
                        fb = eb if fb is None else fb + eb
                    out_v[sel, tl, pl.ds(gr * 16, 16)] = fa
                    out_v[sel, tl, pl.ds(128 + gr * 16, 16)] = fb

        def cbase_of(c):
            return pl.multiple_of(wid * cpw + c * CHUNK, CHUNK)

        def irow_of(c):
            return pl.multiple_of(b_off * K // GROWS
                                  + wid * (cpw * K // GROWS) + c * NBATCH,
                                  NBATCH)

        def stage_copies(c, sel):
            return [
                pltpu.make_async_copy(idx_hbm.at[pl.ds(irow_of(c), NBATCH)],
                                      idx_v.at[sel], semS),
                pltpu.make_async_copy(p_hbm.at[pl.ds(cbase_of(c), CHUNK)],
                                      p_v.at[sel], semS),
                pltpu.make_async_copy(slog_hbm.at[pl.ds(cbase_of(c), CHUNK)],
                                      slog_v.at[sel], semS),
            ]

        def out_copy(c, sel):
            return pltpu.make_async_copy(
                out_v.at[sel], wn_hbm.at[pl.ds(cbase_of(c), CHUNK)], semO)

        H = GROWS // 2

        def gcopies(s, g, buf, sem):
            return [
                pltpu.make_async_copy(
                    table_hbm.at[idx_v.at[s, g, pl.ds(0, H)]],
                    buf.at[pl.ds(0, H)], sem),
                pltpu.make_async_copy(
                    table_hbm.at[idx_v.at[s, g, pl.ds(H, H)]],
                    buf.at[pl.ds(H, H)], sem),
            ]

        def gstart(s, g, buf, sem):
            for cp_ in gcopies(s, g, buf, sem):
                cp_.start()

        def gwait(s, g, buf, sem):
            for cp_ in gcopies(s, g, buf, sem):
                cp_.wait()

        for cp_ in stage_copies(0, 0):
            cp_.start()
        for cp_ in stage_copies(0, 0):
            cp_.wait()
        gstart(0, 0, bufA, semA)

        @pl.loop(0, nchunk)
        def _chunk(c):
            sel = lax.rem(c, 2)
            nsel = 1 - sel

            @pl.when(c + 1 < nchunk)
            def _():
                for cp_ in stage_copies(c + 1, nsel):
                    cp_.start()

            @pl.when(c >= 2)
            def _():
                out_copy(c - 2, sel).wait()

            @pl.loop(0, NBATCH, step=2)
            def _g(g):
                gstart(sel, g + 1, bufB, semB)
                gwait(sel, g, bufA, semA)
                compute_batch(g, bufA, sel)

                @pl.when(g + 2 < NBATCH)
                def _():
                    gstart(sel, g + 2, bufA, semA)

                gwait(sel, g + 1, bufB, semB)
                compute_batch(g + 1, bufB, sel)

            @pl.when(c + 1 < nchunk)
            def _():
                for cp_ in stage_copies(c + 1, nsel):
                    cp_.wait()
                gstart(nsel, 0, bufA, semA)

            out_copy(c, sel).start()

        out_copy(nchunk - 2, 0).wait()
        out_copy(nchunk - 1, 1).wait()

    return sc_kernel(table, idx2, p, slog)


def kernel(center_emb, all_embs, neighbor_indices, neighbor_weights, Wq, Wk, Wg, bg):
    scale = A ** (-0.5)
    wkt = (Wk.T * scale).astype(jnp.float32)
    wg1 = Wg[:D]
    wg2 = Wg[D:]
    bg2 = bg.reshape(1, D)
    idx2 = neighbor_indices.astype(jnp.int32).reshape(B * K // GROWS, GROWS)

    nb = 4000
    bb = 2048
    npk = N // nb
    table, p, slog = pl.pallas_call(
        _tc_packpre_body,
        grid=(npk + B // bb,),
        in_specs=[
            pl.BlockSpec((nb, D), lambda i: (jnp.minimum(i, npk - 1), 0)),
            pl.BlockSpec((bb, D), lambda i: (jnp.maximum(i - npk, 0), 0)),
            pl.BlockSpec((bb, K), lambda i: (jnp.maximum(i - npk, 0), 0)),
            pl.BlockSpec((D, A), lambda i: (0, 0)),
            pl.BlockSpec((A, D), lambda i: (0, 0)),
        ],
        out_specs=[
            pl.BlockSpec((nb, 128), lambda i: (jnp.minimum(i, npk - 1), 0)),
            pl.BlockSpec((bb, 128), lambda i: (jnp.maximum(i - npk, 0), 0)),
            pl.BlockSpec((bb, K), lambda i: (jnp.maximum(i - npk, 0), 0)),
        ],
        out_shape=[
            jax.ShapeDtypeStruct((N, 128), jnp.int32),
            jax.ShapeDtypeStruct((B, 128), jnp.int32),
            jax.ShapeDtypeStruct((B, K), jnp.float32),
        ],
    )(all_embs, center_emb, neighbor_weights, Wq, wkt)

    wn = _sc_attention(table, idx2, p, slog, 0, B)

    out = pl.pallas_call(
        _tc_post_body,
        grid=(B // bb,),
        in_specs=[
            pl.BlockSpec((bb, D), lambda i: (i, 0)),
            pl.BlockSpec((bb, D), lambda i: (i, 0)),
            pl.BlockSpec((D, D), lambda i: (0, 0)),
            pl.BlockSpec((D, D), lambda i: (0, 0)),
            pl.BlockSpec((1, D), lambda i: (0, 0)),
        ],
        out_specs=pl.BlockSpec((bb, D), lambda i: (i, 0)),
        out_shape=jax.ShapeDtypeStruct((B, D), jnp.float32),
    )(center_emb, wn, wg1, wg2, bg2)
    return out


# speedup vs baseline: 1.0028x; 1.0028x over previous
import dataclasses
import functools

import jax
import jax.numpy as jnp
from jax import lax
from jax.experimental import pallas as pl
from jax.experimental.pallas import tpu as pltpu
from jax.experimental.pallas import tpu_sc as plsc

B = 16384
N = 100000
D = 256
K = 16
A = 64

NW = 32
CPW = B // NW
CHUNK = 64
NCHUNK = CPW // CHUNK
GB = 8
GROWS = GB * K
NBATCH = CHUNK // GB

_NEG = -1e30


def _tree_sum(vals):
    vals = list(vals)
    while len(vals) > 1:
        nxt = [vals[i] + vals[i + 1] for i in range(0, len(vals) - 1, 2)]
        if len(vals) % 2:
            nxt.append(vals[-1])
        vals = nxt
    return vals[0]


def _pack_rows(x):
    u = jax.lax.bitcast_convert_type(x, jnp.int32)
    t = u + 0x7FFF + ((u >> 16) & 1)
    bf = t >> 16
    lo = bf[:, :128] & 0xFFFF
    hi = bf[:, 128:] << 16
    return hi | lo


def _tc_pack_body(e_ref, t_ref):
    t_ref[...] = _pack_rows(e_ref[...])


def _tc_pre_body(cb_ref, w_ref, wq_ref, wkt_ref, p_ref, slog_ref):
    q = jnp.dot(cb_ref[...], wq_ref[...], preferred_element_type=jnp.float32)
    p = jnp.dot(q, wkt_ref[...], preferred_element_type=jnp.float32)
    p_ref[...] = _pack_rows(p)
    w = w_ref[...]
    slog_ref[...] = jnp.where(w < 1e-6, _NEG, jnp.log(jnp.maximum(w, 1e-6)))


def _tc_post_body(cb_ref, wn_ref, wg1_ref, wg2_ref, bg_ref, o_ref):
    cb = cb_ref[...]
    wn = wn_ref[...]
    z = (jnp.dot(cb, wg1_ref[...], preferred_element_type=jnp.float32)
         + jnp.dot(wn, wg2_ref[...], preferred_element_type=jnp.float32)
         + bg_ref[...])
    g = jax.nn.sigmoid(z)
    o_ref[...] = g * cb + (1.0 - g) * wn


def _sc_attention(table, idx2, p, slog, b_off, b_cnt):
    cpw = b_cnt // NW
    nchunk = cpw // CHUNK
    mesh = plsc.VectorSubcoreMesh(core_axis_name="c", subcore_axis_name="s")
    cp = pltpu.CompilerParams()
    if "needs_layout_passes" in pltpu.CompilerParams.__dataclass_fields__:
        cp = dataclasses.replace(cp, needs_layout_passes=False)

    @functools.partial(
        pl.kernel,
        out_type=jax.ShapeDtypeStruct((b_cnt, D), jnp.float32),
        mesh=mesh,
        compiler_params=cp,
        scratch_types=[
            pltpu.VMEM((2, NBATCH, GROWS), jnp.int32),
            pltpu.VMEM((2, CHUNK, 128), jnp.int32),
            pltpu.VMEM((2, CHUNK, K), jnp.float32),
            pltpu.VMEM((2, CHUNK, D), jnp.float32),
            pltpu.VMEM((GROWS, 128), jnp.int32),
            pltpu.VMEM((GROWS, 128), jnp.int32),
            pltpu.VMEM((GROWS, 16), jnp.float32),
            pltpu.SemaphoreType.DMA,
            pltpu.SemaphoreType.DMA,
            pltpu.SemaphoreType.DMA,
            pltpu.SemaphoreType.DMA,
        ],
    )
    def sc_kernel(table_hbm, idx_hbm, p_hbm, slog_hbm, wn_hbm,
                  idx_v, p_v, slog_v, out_v, bufA, bufB, acc_v,
                  semA, semB, semS, semO):
        cid = lax.axis_index("c")
        sid = lax.axis_index("s")
        wid = sid * 2 + cid
        lane = lax.broadcasted_iota(jnp.int32, (K,), 0)
        cols = [jnp.full((16,), d_, jnp.int32) for d_ in range(16)]

        def compute_batch(g, buf, sel):
            @plsc.parallel_loop(0, GB, unroll=2)
            def _t(t):
                tl = g * GB + t
                r0 = t * K
                a0 = t * K
                pchb = [plsc.bitcast(p_v[sel, tl, pl.ds(gr * 16, 16)],
                                     jnp.bfloat16) for gr in range(8)]
                for j in range(K):
                    pp = [pchb[gr]
                          * plsc.bitcast(buf[r0 + j, pl.ds(gr * 16, 16)],
                                         jnp.bfloat16)
                          for gr in range(8)]
                    prods = []
                    for q2 in range(4):
                        ea, eb = plsc.unpack(
                            pp[2 * q2] + pp[2 * q2 + 1],
                            format=plsc.PackFormat.INTERLEAVED)
                        prods.append(ea)
                        prods.append(eb)
                    acc_v[a0 + j, :] = _tree_sum(prods)
                tot = _tree_sum(
                    [plsc.load_gather(acc_v, [lane + a0, cols[d_]])
                     for d_ in range(16)])
                s = slog_v[sel, tl, :] + tot
                e = jnp.exp(s)
                den = jnp.sum(e)
                den = jnp.where(den > 0.0, den, 1.0)
                attn = e / den
                ajb = []
                for j in range(K):
                    bcast = jnp.broadcast_to(attn[j], (16,))
                    ajb.append(plsc.pack(bcast, bcast,
                                         format=plsc.PackFormat.INTERLEAVED))
                for gr in range(8):
                    fa, fb = None, None
                    for q4 in range(4):
                        pr = None
                        for j in range(4 * q4, 4 * q4 + 4):
                            term = ajb[j] * plsc.bitcast(
                                buf[r0 + j, pl.ds(gr * 16, 16)],
                                jnp.bfloat16)
                            pr = term if pr is None else pr + term
                        ea, eb = plsc.unpack(
                            pr, format=plsc.PackFormat.INTERLEAVED)
                        fa = ea if fa is None else fa + ea
                        fb = eb if fb is None else fb + eb
                    out_v[sel, tl, pl.ds(gr * 16, 16)] = fa
                    out_v[sel, tl, pl.ds(128 + gr * 16, 16)] = fb

        def cbase_of(c):
            return pl.multiple_of(wid * cpw + c * CHUNK, CHUNK)

        def irow_of(c):
            return pl.multiple_of(b_off * K // GROWS
                                  + wid * (cpw * K // GROWS) + c * NBATCH,
                                  NBATCH)

        def stage_copies(c, sel):
            return [
                pltpu.make_async_copy(idx_hbm.at[pl.ds(irow_of(c), NBATCH)],
                                      idx_v.at[sel], semS),
                pltpu.make_async_copy(p_hbm.at[pl.ds(cbase_of(c), CHUNK)],
                                      p_v.at[sel], semS),
                pltpu.make_async_copy(slog_hbm.at[pl.ds(cbase_of(c), CHUNK)],
                                      slog_v.at[sel], semS),
            ]

        def out_copy(c, sel):
            return pltpu.make_async_copy(
                out_v.at[sel], wn_hbm.at[pl.ds(cbase_of(c), CHUNK)], semO)

        H = GROWS // 2

        def gcopies(s, g, buf, sem):
            return [
                pltpu.make_async_copy(
                    table_hbm.at[idx_v.at[s, g, pl.ds(0, H)]],
                    buf.at[pl.ds(0, H)], sem),
                pltpu.make_async_copy(
                    table_hbm.at[idx_v.at[s, g, pl.ds(H, H)]],
                    buf.at[pl.ds(H, H)], sem),
            ]

        def gstart(s, g, buf, sem):
            for cp_ in gcopies(s, g, buf, sem):
                cp_.start()

        def gwait(s, g, buf, sem):
            for cp_ in gcopies(s, g, buf, sem):
                cp_.wait()

        for cp_ in stage_copies(0, 0):
            cp_.start()
        for cp_ in stage_copies(0, 0):
            cp_.wait()
        gstart(0, 0, bufA, semA)

        @pl.loop(0, nchunk)
        def _chunk(c):
            sel = lax.rem(c, 2)
            nsel = 1 - sel

            @pl.when(c + 1 < nchunk)
            def _():
                for cp_ in stage_copies(c + 1, nsel):
                    cp_.start()

            @pl.when(c >= 2)
            def _():
                out_copy(c - 2, sel).wait()

            @pl.loop(0, NBATCH, step=2)
            def _g(g):
                gstart(sel, g + 1, bufB, semB)
                gwait(sel, g, bufA, semA)
                compute_batch(g, bufA, sel)

                @pl.when(g + 2 < NBATCH)
                def _():
                    gstart(sel, g + 2, bufA, semA)

                gwait(sel, g + 1, bufB, semB)
                compute_batch(g + 1, bufB, sel)

            @pl.when(c + 1 < nchunk)
            def _():
                for cp_ in stage_copies(c + 1, nsel):
                    cp_.wait()
                gstart(nsel, 0, bufA, semA)

            out_copy(c, sel).start()

        out_copy(nchunk - 2, 0).wait()
        out_copy(nchunk - 1, 1).wait()

    return sc_kernel(table, idx2, p, slog)


def kernel(center_emb, all_embs, neighbor_indices, neighbor_weights, Wq, Wk, Wg, bg):
    scale = A ** (-0.5)
    wkt = (Wk.T * scale).astype(jnp.float32)
    wg1 = Wg[:D]
    wg2 = Wg[D:]
    bg2 = bg.reshape(1, D)
    idx2 = neighbor_indices.astype(jnp.int32).reshape(B * K // GROWS, GROWS)

    nb = 4000
    table = pl.pallas_call(
        _tc_pack_body,
        grid=(N // nb,),
        in_specs=[pl.BlockSpec((nb, D), lambda i: (i, 0))],
        out_specs=pl.BlockSpec((nb, 128), lambda i: (i, 0)),
        out_shape=jax.ShapeDtypeStruct((N, 128), jnp.int32),
    )(all_embs)

    bb = 2048
    p, slog = pl.pallas_call(
        _tc_pre_body,
        grid=(B // bb,),
        in_specs=[
            pl.BlockSpec((bb, D), lambda i: (i, 0)),
            pl.BlockSpec((bb, K), lambda i: (i, 0)),
            pl.BlockSpec((D, A), lambda i: (0, 0)),
            pl.BlockSpec((A, D), lambda i: (0, 0)),
        ],
        out_specs=[
            pl.BlockSpec((bb, 128), lambda i: (i, 0)),
            pl.BlockSpec((bb, K), lambda i: (i, 0)),
        ],
        out_shape=[
            jax.ShapeDtypeStruct((B, 128), jnp.int32),
            jax.ShapeDtypeStruct((B, K), jnp.float32),
        ],
    )(center_emb, neighbor_weights, Wq, wkt)

    wn = _sc_attention(table, idx2, p, slog, 0, B)

    out = pl.pallas_call(
        _tc_post_body,
        grid=(B // bb,),
        in_specs=[
            pl.BlockSpec((bb, D), lambda i: (i, 0)),
            pl.BlockSpec((bb, D), lambda i: (i, 0)),
            pl.BlockSpec((D, D), lambda i: (0, 0)),
            pl.BlockSpec((D, D), lambda i: (0, 0)),
            pl.BlockSpec((1, D), lambda i: (0, 0)),
        ],
        out_specs=pl.BlockSpec((bb, D), lambda i: (i, 0)),
        out_shape=jax.ShapeDtypeStruct((B, D), jnp.float32),
    )(center_emb, wn, wg1, wg2, bg2)
    return out
